# baseline (speedup 1.0000x reference)
import numpy as np
import jax
import jax.numpy as jnp
from jax import lax
from jax.experimental import pallas as pl
from jax.experimental.pallas import tpu as pltpu

B = 4096
F = 26
K = 64
D = F * K
T = 16
H = 256
BB = 128

C0 = float(np.log(np.float32(1e-30)))
LOGK = float(np.log(np.float32(K)))


def _np_lae(a, b):
    m = np.maximum(a, b)
    return (m + np.log(np.exp(a - m) + np.exp(b - m))).astype(np.float32)


def _tables():
    beta = np.linspace(1e-4, 0.02, T)
    alpha = 1.0 - beta
    cum = np.cumprod(alpha)
    a1 = np.log(alpha).astype(np.float32)
    b1_ = np.log(1.0 - alpha).astype(np.float32)
    ac = np.log(cum).astype(np.float32)
    bc = np.log(1.0 - cum).astype(np.float32)
    ph = _np_lae(ac, bc - LOGK)
    pm = _np_lae(np.float32(C0) + ac, bc - LOGK)
    qh = _np_lae(a1, b1_ - LOGK)
    qm = _np_lae(np.float32(C0) + a1, b1_ - LOGK)
    ts = np.arange(T)
    eh = np.where(ts == 0, np.float32(0.0), np.concatenate([[0.0], ph[:-1]]).astype(np.float32))
    em = np.where(ts == 0, np.float32(C0), np.concatenate([[C0], pm[:-1]]).astype(np.float32))
    tm1 = np.maximum(ts - 1, 0)
    a_c = ac[tm1]
    b_c = bc[tm1] - LOGK
    is0 = (ts == 0).astype(np.float32)
    cols = [ph, pm, qh, qm, eh, em, a_c, b_c, is0]
    tbl = np.zeros((T, 16), dtype=np.float32)
    for j, c in enumerate(cols):
        tbl[:, j] = c
    return tbl, float(ph[T - 1]), float(pm[T - 1])


TBL_NP, PHT, PMT = _tables()


def _body(u_ref, x0_ref, toh_ref, w1_ref, w2_ref, b1_ref, b2_ref, temb_ref,
          tbl_ref, out_ref, pred_scr):
    u = u_ref[...]
    x0 = x0_ref[...]
    toh = toh_ref[...]
    cons = jnp.dot(toh, tbl_ref[...], preferred_element_type=jnp.float32)

    def col(j):
        return cons[:, j:j + 1][:, :, None]

    ph_s, pm_s, qh, qm, eh, em, a_c, b_c, is0 = (col(j) for j in range(9))

    kio = lax.broadcasted_iota(jnp.int32, (BB, F, K), 2)
    oh0 = kio == x0[:, :, None]

    g = -jnp.log(-jnp.log(u + 1e-30) + 1e-30)
    sl = g + jnp.where(oh0, ph_s, pm_s)
    msl = jnp.max(sl, axis=2, keepdims=True)
    xt = jnp.min(jnp.where(sl == msl, kio, K), axis=2)
    oht = kio == xt[:, :, None]

    lxt = jnp.where(oht, jnp.float32(0.0), jnp.float32(C0))
    h = jnp.dot(toh, temb_ref[...], preferred_element_type=jnp.float32)
    for f in range(F):
        h = h + jnp.dot(lxt[:, f, :], w1_ref[f], preferred_element_type=jnp.float32)
    h = jnp.maximum(h + b1_ref[...], 0.0)
    for f in range(F):
        pred_scr[:, f, :] = (jnp.dot(h, w2_ref[f], preferred_element_type=jnp.float32)
                             + b2_ref[f])
    pred = pred_scr[...]

    m1 = jnp.max(pred, axis=2, keepdims=True)
    lsm = pred - (m1 + jnp.log(jnp.sum(jnp.exp(pred - m1), axis=2, keepdims=True)))

    a_ = lsm + a_c
    mm = jnp.maximum(a_, b_c)
    lae = mm + jnp.log(jnp.exp(a_ - mm) + jnp.exp(b_c - mm))
    ev = jnp.where(is0 > 0.5, lsm, lae)
    qterm = jnp.where(oht, qh, qm)
    une = ev + qterm
    m2 = jnp.max(une, axis=2, keepdims=True)
    log_est = une - (m2 + jnp.log(jnp.sum(jnp.exp(une - m2), axis=2, keepdims=True)))

    unt = jnp.where(oh0, eh, em) + qterm
    m3 = jnp.max(unt, axis=2, keepdims=True)
    log_true = unt - (m3 + jnp.log(jnp.sum(jnp.exp(unt - m3), axis=2, keepdims=True)))

    kl = jnp.sum(jnp.exp(log_true) * (log_true - log_est), axis=2)
    lx0 = jnp.where(oh0, jnp.float32(0.0), jnp.float32(C0))
    nll = -jnp.sum(jnp.exp(lx0) * log_est, axis=2)
    dl = jnp.where(cons[:, 8:9] > 0.5, nll, kl)

    vT = jnp.where(oh0, jnp.float32(PHT), jnp.float32(PMT))
    recon = jnp.exp(vT) * (vT + LOGK)

    total = (jnp.sum(dl) * jnp.float32(T) + jnp.sum(recon)).reshape(1, 1)

    @pl.when(pl.program_id(0) == 0)
    def _init():
        out_ref[...] = jnp.zeros((1, 1), jnp.float32)
    out_ref[...] += total


def kernel(W1, b1, W2, b2, temb, x_0):
    kt = jax.random.key(1)
    t = jax.random.randint(jax.random.fold_in(kt, 0), (B,), 0, T)
    u = jax.random.uniform(jax.random.fold_in(kt, 1), (B, F, K))
    toh = jax.nn.one_hot(t, T, dtype=jnp.float32)
    x0i = x_0.astype(jnp.int32)
    w1r = W1.reshape(F, K, H)
    w2r = W2.reshape(H, F, K).transpose(1, 0, 2)
    b2r = b2.reshape(F, 1, K)
    tbl = jnp.asarray(TBL_NP)
    out = pl.pallas_call(
        _body,
        grid=(B // BB,),
        in_specs=[
            pl.BlockSpec((BB, F, K), lambda i: (i, 0, 0)),
            pl.BlockSpec((BB, F), lambda i: (i, 0)),
            pl.BlockSpec((BB, T), lambda i: (i, 0)),
            pl.BlockSpec((F, K, H), lambda i: (0, 0, 0)),
            pl.BlockSpec((F, H, K), lambda i: (0, 0, 0)),
            pl.BlockSpec((1, H), lambda i: (0, 0)),
            pl.BlockSpec((F, 1, K), lambda i: (0, 0, 0)),
            pl.BlockSpec((T, H), lambda i: (0, 0)),
            pl.BlockSpec((T, 16), lambda i: (0, 0)),
        ],
        out_specs=pl.BlockSpec((1, 1), lambda i: (0, 0)),
        out_shape=jax.ShapeDtypeStruct((1, 1), jnp.float32),
        scratch_shapes=[pltpu.VMEM((BB, F, K), jnp.float32)],
    )(u, x0i, toh, w1r, w2r, b1.reshape(1, H), b2r, temb, tbl)
    return out[0, 0] / B
